# baseline (speedup 1.0000x reference)
import functools
import math

import jax
import jax.numpy as jnp
import numpy as np
from jax import lax
from jax.experimental import pallas as pl
from jax.experimental.pallas import tpu as pltpu
from jax.experimental.pallas import tpu_sc as plsc

D_MODEL = 64
SEQ = 50
LANES = 16
NUM_WORKERS = 32


def _pos_encoding(length: int, depth: int) -> np.ndarray:
    half = depth / 2
    positions = np.arange(length)[:, np.newaxis]
    depths = np.arange(half)[np.newaxis, :] / half
    angle_rates = 1 / 10000**depths
    angle_rads = positions * angle_rates
    return np.concatenate(
        [np.sin(angle_rads), np.cos(angle_rads)], axis=-1
    ).astype(np.float32)


@functools.partial(jax.jit, static_argnames=("flat", "chunk"))
def _embed_sc(table, idx, pos, *, flat: int, chunk: int):
    rows_per_worker = flat // NUM_WORKERS
    n_chunks = rows_per_worker // chunk
    d_blocks = D_MODEL // LANES
    groups = chunk // SEQ

    mesh = plsc.VectorSubcoreMesh(core_axis_name="c", subcore_axis_name="s")

    @functools.partial(
        pl.kernel,
        out_type=jax.ShapeDtypeStruct((flat, D_MODEL), jnp.float32),
        mesh=mesh,
        scratch_types=[
            pltpu.VMEM((chunk,), jnp.int32),
            pltpu.VMEM((chunk, D_MODEL), jnp.float32),
            pltpu.VMEM((SEQ, D_MODEL), jnp.float32),
            pltpu.SemaphoreType.DMA,
        ],
        compiler_params=pltpu.CompilerParams(use_tc_tiling_on_sc=False),
    )
    def body(table_hbm, idx_hbm, pos_hbm, out_hbm, idx_v, rows_v, pos_v, sem):
        wid = lax.axis_index("s") * 2 + lax.axis_index("c")
        base = wid * rows_per_worker
        pltpu.sync_copy(pos_hbm, pos_v)

        for c in range(n_chunks):
            start = base + c * chunk
            pltpu.sync_copy(idx_hbm.at[pl.ds(start, chunk)], idx_v)
            pltpu.async_copy(table_hbm.at[idx_v], rows_v, sem).wait()

            @pl.loop(0, SEQ)
            def _(p):
                for d in range(d_blocks):
                    pe = pos_v[p, pl.ds(d * LANES, LANES)]
                    for g in range(groups):
                        j = p + g * SEQ
                        v = rows_v[j, pl.ds(d * LANES, LANES)]
                        rows_v[j, pl.ds(d * LANES, LANES)] = v * 8.0 + pe

            pltpu.sync_copy(rows_v, out_hbm.at[pl.ds(start, chunk)])

    return body(table, idx, pos)


_POS = _pos_encoding(SEQ, D_MODEL)


def kernel(x, table):
    batch, seq = x.shape
    assert seq == SEQ and table.shape[1] == D_MODEL
    flat = batch * seq
    idx = x.reshape(flat).astype(jnp.int32)
    pos = jnp.asarray(_POS)
    out = _embed_sc(table, idx, pos, flat=flat, chunk=800)
    return out.reshape(batch, seq, D_MODEL)
